# baseline (speedup 1.0000x reference)
import functools

import jax
import jax.numpy as jnp
from jax import lax
from jax.experimental import pallas as pl
from jax.experimental.pallas import tpu as pltpu
from jax.experimental.pallas import tpu_sc as plsc

NC = 2
NS = 16
LN = 16
NG = 64


def _edge_embed_body(ea_ref, we_ref, be_ref, out_ref):
    w = out_ref.shape[2]
    res = jnp.dot(ea_ref[...], we_ref[...],
                  preferred_element_type=jnp.float32) + be_ref[...]
    out_ref[0] = res[:, :w]
    out_ref[1] = res[:, w:]


def _edge_embed(edge_attr, we, be, blk=None):
    e, ed = edge_attr.shape
    blk = blk or (2000 if e % 2000 == 0 else e)
    di = we.shape[1]
    w = di // 2
    out = pl.pallas_call(
        _edge_embed_body,
        grid=(e // blk,),
        in_specs=[
            pl.BlockSpec((blk, ed), lambda i: (i, 0)),
            pl.BlockSpec((ed, di), lambda i: (0, 0)),
            pl.BlockSpec((1, di), lambda i: (0, 0)),
        ],
        out_specs=pl.BlockSpec((2, blk, w), lambda i: (0, i, 0)),
        out_shape=jax.ShapeDtypeStruct((2, e, w), jnp.float32),
    )(edge_attr, we, be.reshape(1, di))
    return out.reshape(2 * e, w)


_FB = 2304
_BS = 2000


def _bucket_cap(e):
    return e // 2 + 4096


def _make_bucket(n, e):
    npad = -(-n // (NS * 8)) * (NS * 8)
    npt = npad // NS
    half = e // 2
    cap = _bucket_cap(e)
    nog = half // _BS
    mesh = plsc.VectorSubcoreMesh(core_axis_name="c", subcore_axis_name="s",
                                  num_cores=NC, num_subcores=NS)

    @functools.partial(
        pl.kernel,
        out_type=[
            jax.ShapeDtypeStruct((2 * NS * cap,), jnp.int32),
            jax.ShapeDtypeStruct((2 * NS * cap,), jnp.int32),
            jax.ShapeDtypeStruct((2 * NS * cap,), jnp.int32),
            jax.ShapeDtypeStruct((2 * NS, LN), jnp.int32),
        ],
        mesh=mesh,
        scratch_types=[
            pltpu.VMEM((2, _BS), jnp.int32),
            pltpu.VMEM((2, _BS), jnp.int32),
            pltpu.VMEM((_FB,), jnp.int32),
            pltpu.VMEM((_FB,), jnp.int32),
            pltpu.VMEM((_FB,), jnp.int32),
            pltpu.VMEM((LN,), jnp.int32),
            pltpu.SemaphoreType.DMA,
            pltpu.SemaphoreType.DMA,
        ],
        compiler_params=pltpu.CompilerParams(use_tc_tiling_on_sc=False,
                                             needs_layout_passes=False),
    )
    def bucket(srch, dsth, eido, srco, dsto, cnto,
               sbuf, dbuf, eidl, srcl, dstl, cbuf, sem0, sem1):
        cid = lax.axis_index("c")
        sid = lax.axis_index("s")
        lo = sid * npt
        hi = lo + npt

        ebase = cid * half
        region = (cid * NS + sid) * cap
        iota = lax.iota(jnp.int32, LN)

        def flush3(written):
            off = pl.multiple_of(region + written, 2048)
            pltpu.sync_copy(eidl.at[pl.ds(0, 2048)],
                            eido.at[pl.ds(off, 2048)])
            pltpu.sync_copy(srcl.at[pl.ds(0, 2048)],
                            srco.at[pl.ds(off, 2048)])
            pltpu.sync_copy(dstl.at[pl.ds(0, 2048)],
                            dsto.at[pl.ds(off, 2048)])

        sems = (sem0, sem1)

        def issue_group(g, p):
            gb = ebase + g * _BS
            pltpu.async_copy(srch.at[pl.ds(gb, _BS)], sbuf.at[p], sems[p])
            pltpu.async_copy(dsth.at[pl.ds(gb, _BS)], dbuf.at[p], sems[p])

        def wait_group(p):
            pltpu.make_async_copy(srch.at[pl.ds(0, _BS)], sbuf.at[p],
                                  sems[p]).wait()
            pltpu.make_async_copy(dsth.at[pl.ds(0, _BS)], dbuf.at[p],
                                  sems[p]).wait()

        def group(g, carry, p):
            cnt, written = carry
            gb = ebase + g * _BS
            wait_group(p)

            def step(i, carry2):
                cnt2, written2 = carry2
                vd = dbuf[p, pl.ds(i * LN, LN)]
                vs = sbuf[p, pl.ds(i * LN, LN)]
                m = jnp.logical_and(vd >= lo, vd < hi)
                pc = plsc.cumsum(m.astype(jnp.int32))
                pos = cnt2 + pc - 1
                eidv = gb + i * LN + iota
                plsc.store_scatter(eidl, [pos], eidv, mask=m)
                plsc.store_scatter(srcl, [pos], vs, mask=m)
                plsc.store_scatter(dstl, [pos], vd - lo, mask=m)
                cnt2 = cnt2 + pc[LN - 1]

                @pl.when(cnt2 >= 2048)
                def _():
                    flush3(written2)
                    eidl[pl.ds(0, LN)] = eidl[pl.ds(2048, LN)]
                    srcl[pl.ds(0, LN)] = srcl[pl.ds(2048, LN)]
                    dstl[pl.ds(0, LN)] = dstl[pl.ds(2048, LN)]

                spill = cnt2 >= 2048
                cnt2 = jnp.where(spill, cnt2 - 2048, cnt2)
                written2 = jnp.where(spill, written2 + 2048, written2)
                return (cnt2, written2)

            carry = lax.fori_loop(0, _BS // LN, step, (cnt, written))

            @pl.when(g + 2 < nog)
            def _():
                issue_group(g + 2, p)
            return carry

        def gpair(jj, carry):
            carry = group(2 * jj, carry, 0)
            return group(2 * jj + 1, carry, 1)

        issue_group(0, 0)

        @pl.when(nog > 1)
        def _():
            issue_group(1, 1)
        cnt, written = lax.fori_loop(0, nog // 2, gpair, (0, 0))
        dummy_d = jnp.full((LN,), npt, jnp.int32)
        zero_v = jnp.zeros((LN,), jnp.int32)
        for j in range(128 // LN):
            off = cnt + j * LN + iota
            plsc.store_scatter(eidl, [off], zero_v)
            plsc.store_scatter(srcl, [off], zero_v)
            plsc.store_scatter(dstl, [off], dummy_d)
        flush3(written)

        @pl.when(cnt + 128 > 2048)
        def _():
            off2 = pl.multiple_of(region + written + 2048, 2048)
            pltpu.sync_copy(eidl.at[pl.ds(2048, 256)],
                            eido.at[pl.ds(off2, 256)])
            pltpu.sync_copy(srcl.at[pl.ds(2048, 256)],
                            srco.at[pl.ds(off2, 256)])
            pltpu.sync_copy(dstl.at[pl.ds(2048, 256)],
                            dsto.at[pl.ds(off2, 256)])

        cbuf[pl.ds(0, LN)] = jnp.full((LN,), written + cnt, jnp.int32)
        pltpu.sync_copy(cbuf, cnto.at[cid * NS + sid])

    return bucket


def _make_mp(n, e, w):
    npad = -(-n // (NS * 8)) * (NS * 8)
    npt = npad // NS
    stride = npt + 8
    cap = _bucket_cap(e)
    _CH = 80 if w >= 128 else 128
    mesh = plsc.VectorSubcoreMesh(core_axis_name="c", subcore_axis_name="s",
                                  num_cores=NC, num_subcores=NS)

    @functools.partial(
        pl.kernel,
        out_type=jax.ShapeDtypeStruct((NC, npad, w), jnp.float32),
        mesh=mesh,
        scratch_types=[
            pltpu.VMEM((2, _CH), jnp.int32),
            pltpu.VMEM((2, _CH), jnp.int32),
            pltpu.VMEM((2, _CH), jnp.int32),
            pltpu.VMEM((2, _CH), jnp.int32),
            pltpu.VMEM((2, LN), jnp.int32),
            pltpu.VMEM((2, _CH, w), jnp.float32),
            pltpu.VMEM((2, _CH, w), jnp.float32),
            pltpu.VMEM_SHARED((NS * (npt + 8), w), jnp.float32),
            pltpu.SemaphoreType.DMA,
            pltpu.SemaphoreType.DMA,
            pltpu.SemaphoreType.DMA,
            pltpu.SemaphoreType.DMA,
            pltpu.SemaphoreType.DMA,
        ],
        compiler_params=pltpu.CompilerParams(use_tc_tiling_on_sc=False),
    )
    def mp(z2, e2, eidh, srch, dsth, cnth, zeros, out,
           eidb, srcb, dstb, dsts, cbuf, zbuf, ebuf, acc,
           semi0, semi1, semg0, semg1, sems):
        cid = lax.axis_index("c")
        sid = lax.axis_index("s")
        arow = sid * stride
        pltpu.sync_copy(zeros, acc.at[pl.ds(arow, stride)])
        zoff = cid * n
        eoff = cid * e

        pltpu.sync_copy(cnth.at[sid], cbuf.at[0])
        pltpu.sync_copy(cnth.at[NS + sid], cbuf.at[1])
        cnt0 = cbuf[0, pl.ds(0, LN)][0]
        cnt1 = cbuf[1, pl.ds(0, LN)][0]
        nch0 = (cnt0 + (_CH - 1)) // _CH
        nch1 = (cnt1 + (_CH - 1)) // _CH
        total = nch0 + nch1
        base0 = sid * cap
        base1 = (NS + sid) * cap
        semi = (semi0, semi1)
        semg = (semg0, semg1)

        def off_of(ci):
            return jnp.where(ci < nch0, base0 + ci * _CH,
                             base1 + (ci - nch0) * _CH)

        def issue_idx(ci, p):
            o = pl.multiple_of(off_of(ci), 8)
            pltpu.async_copy(eidh.at[pl.ds(o, _CH)], eidb.at[p], semi[p])
            pltpu.async_copy(srch.at[pl.ds(o, _CH)], srcb.at[p], semi[p])
            pltpu.async_copy(dsth.at[pl.ds(o, _CH)], dstb.at[p], semi[p])

        def wait_idx(p):
            pltpu.make_async_copy(eidh.at[pl.ds(0, _CH)], eidb.at[p],
                                  semi[p]).wait()
            pltpu.make_async_copy(srch.at[pl.ds(0, _CH)], srcb.at[p],
                                  semi[p]).wait()
            pltpu.make_async_copy(dsth.at[pl.ds(0, _CH)], dstb.at[p],
                                  semi[p]).wait()

        def adjust_and_gather(p):
            for k in range(_CH // LN):
                sl = pl.ds(k * LN, LN)
                srcb[p, sl] = srcb[p, sl] + zoff
                eidb[p, sl] = eidb[p, sl] + eoff
            pltpu.async_copy(z2.at[srcb.at[p]], zbuf.at[p], semg[p])
            pltpu.async_copy(e2.at[eidb.at[p]], ebuf.at[p], semg[p])

        def wait_gather(p):
            pltpu.make_async_copy(z2.at[pl.ds(0, _CH)], zbuf.at[p],
                                  semg[p]).wait()
            pltpu.make_async_copy(e2.at[pl.ds(0, _CH)], ebuf.at[p],
                                  semg[p]).wait()

        def compute(p):
            def rowfn(rr, rc):
                for k in range(w // LN):
                    sl = pl.ds(k * LN, LN)
                    v = zbuf[p, rr, sl] + ebuf[p, rr, sl]
                    ebuf[p, rr, sl] = jnp.maximum(v, 0.0)
                return rc
            lax.fori_loop(0, _CH, rowfn, 0)

        def save_dst(p):
            for k in range(_CH // LN):
                sl = pl.ds(k * LN, LN)
                dsts[p, sl] = dstb[p, sl] + arow

        def issue_scatter(p):
            pltpu.async_copy(ebuf.at[p], acc.at[dsts.at[p]], sems, add=True)

        def drain_scatter():
            pltpu.make_async_copy(ebuf.at[0], acc.at[pl.ds(0, _CH)],
                                  sems).wait()

        @pl.when(total > 0)
        def _():
            issue_idx(0, 0)
            wait_idx(0)
            adjust_and_gather(0)

            @pl.when(total > 1)
            def _():
                issue_idx(1, 1)

            def pair(j, carry):
                i0 = 2 * j

                def stage(i, p, q):
                    @pl.when(i >= 1)
                    def _():
                        drain_scatter()

                    @pl.when(i + 1 < total)
                    def _():
                        wait_idx(q)
                        adjust_and_gather(q)
                    wait_gather(p)
                    save_dst(p)

                    @pl.when(i + 2 < total)
                    def _():
                        issue_idx(i + 2, p)
                    compute(p)
                    issue_scatter(p)

                @pl.when(i0 < total)
                def _():
                    stage(i0, 0, 1)

                @pl.when(i0 + 1 < total)
                def _():
                    stage(i0 + 1, 1, 0)
                return carry

            lax.fori_loop(0, (total + 1) // 2, pair, 0)
            drain_scatter()

        pltpu.sync_copy(acc.at[pl.ds(arow, npt)],
                        out.at[cid, pl.ds(sid * npt, npt)])

    return mp


def _mlp_body(eps_ref, z2_ref, agg2_ref, w1_ref, b1_ref, w2_ref, b2_ref,
              batch_ref, p2_ref, sums_ref, ssq_ref, sp_ref, cnt_ref):
    i = pl.program_id(0)
    wo = p2_ref.shape[2]
    z = jnp.concatenate([z2_ref[0], z2_ref[1]], axis=1)
    agg = jnp.concatenate([agg2_ref[0], agg2_ref[1]], axis=1)
    h = (1.0 + eps_ref[0]) * z + agg
    h = jnp.maximum(
        jnp.dot(h, w1_ref[...], preferred_element_type=jnp.float32)
        + b1_ref[...], 0.0)
    h = jnp.dot(h, w2_ref[...], preferred_element_type=jnp.float32) + b2_ref[...]
    p = jnp.maximum(h, 0.0)
    p2_ref[0] = p[:, :wo]
    p2_ref[1] = p[:, wo:]

    blk = z.shape[0]
    bvec = batch_ref[0]
    giota = lax.broadcasted_iota(jnp.int32, (NG, blk), 0)
    onehot = (giota == bvec).astype(jnp.float32)

    @pl.when(i == 0)
    def _():
        sums_ref[...] = jnp.zeros_like(sums_ref)
        ssq_ref[...] = jnp.zeros_like(ssq_ref)
        sp_ref[...] = jnp.zeros_like(sp_ref)
        cnt_ref[...] = jnp.zeros_like(cnt_ref)

    do = p.shape[1]
    sums_ref[...] += jnp.broadcast_to(jnp.sum(p, 0, keepdims=True), (8, do))
    ssq_ref[...] += jnp.broadcast_to(jnp.sum(p * p, 0, keepdims=True), (8, do))
    sp_ref[...] += jnp.dot(onehot, p, preferred_element_type=jnp.float32)
    cnt_ref[...] += jnp.broadcast_to(
        jnp.sum(onehot, 1, keepdims=True), (NG, do))


def _mlp_call(eps, z2, agg2, w1, b1, w2, b2, batch3, blk=2000):
    n = z2.shape[1]
    wz = z2.shape[2]
    do = w1.shape[1]
    wo = do // 2
    di = w1.shape[0]
    grid = (n // blk,)
    return pl.pallas_call(
        _mlp_body,
        grid=grid,
        in_specs=[
            pl.BlockSpec(memory_space=pltpu.SMEM),
            pl.BlockSpec((2, blk, wz), lambda i: (0, i, 0)),
            pl.BlockSpec((2, blk, wz), lambda i: (0, i, 0)),
            pl.BlockSpec((di, do), lambda i: (0, 0)),
            pl.BlockSpec((1, do), lambda i: (0, 0)),
            pl.BlockSpec((do, do), lambda i: (0, 0)),
            pl.BlockSpec((1, do), lambda i: (0, 0)),
            pl.BlockSpec((1, 1, blk), lambda i: (i, 0, 0)),
        ],
        out_specs=[
            pl.BlockSpec((2, blk, wo), lambda i: (0, i, 0)),
            pl.BlockSpec((8, do), lambda i: (0, 0)),
            pl.BlockSpec((8, do), lambda i: (0, 0)),
            pl.BlockSpec((NG, do), lambda i: (0, 0)),
            pl.BlockSpec((NG, do), lambda i: (0, 0)),
        ],
        out_shape=[
            jax.ShapeDtypeStruct((2, n, wo), jnp.float32),
            jax.ShapeDtypeStruct((8, do), jnp.float32),
            jax.ShapeDtypeStruct((8, do), jnp.float32),
            jax.ShapeDtypeStruct((NG, do), jnp.float32),
            jax.ShapeDtypeStruct((NG, do), jnp.float32),
        ],
    )(eps, z2, agg2, w1, b1.reshape(1, do), w2, b2.reshape(1, do), batch3)


def _bn_body(n_ref, p2_ref, sums_ref, ssq_ref, gamma_ref, beta_ref,
             z2_ref, zf_ref):
    wo = p2_ref.shape[2]
    p = jnp.concatenate([p2_ref[0], p2_ref[1]], axis=1)
    n = n_ref[0]
    mean = sums_ref[0:1] / n
    var = ssq_ref[0:1] / n - mean * mean
    scale = lax.rsqrt(var + 1e-5) * gamma_ref[...]
    zz = (p - mean) * scale + beta_ref[...]
    z2_ref[0] = zz[:, :wo]
    z2_ref[1] = zz[:, wo:]
    zf_ref[...] = zz


def _bn_call(nvec, p2, sums, ssq, gamma, beta, blk=2000):
    n = p2.shape[1]
    wo = p2.shape[2]
    do = 2 * wo
    return pl.pallas_call(
        _bn_body,
        grid=(n // blk,),
        in_specs=[
            pl.BlockSpec(memory_space=pltpu.SMEM),
            pl.BlockSpec((2, blk, wo), lambda i: (0, i, 0)),
            pl.BlockSpec((8, do), lambda i: (0, 0)),
            pl.BlockSpec((8, do), lambda i: (0, 0)),
            pl.BlockSpec((1, do), lambda i: (0, 0)),
            pl.BlockSpec((1, do), lambda i: (0, 0)),
        ],
        out_specs=[
            pl.BlockSpec((2, blk, wo), lambda i: (0, i, 0)),
            pl.BlockSpec((blk, do), lambda i: (i, 0)),
        ],
        out_shape=[
            jax.ShapeDtypeStruct((2, n, wo), jnp.float32),
            jax.ShapeDtypeStruct((n, do), jnp.float32),
        ],
    )(nvec, p2, sums, ssq, gamma.reshape(1, do), beta.reshape(1, do))


def _g_body(n_ref, sp_ref, cnt_ref, sums_ref, ssq_ref, gamma_ref, beta_ref,
            g_ref):
    n = n_ref[0]
    mean = sums_ref[0:1] / n
    var = ssq_ref[0:1] / n - mean * mean
    scale = lax.rsqrt(var + 1e-5) * gamma_ref[...]
    cnt = cnt_ref[...]
    g_ref[...] = (sp_ref[...] - cnt * mean) * scale + cnt * beta_ref[...]


def _g_call(nvec, sp, cnt, sums, ssq, gamma, beta):
    do = sp.shape[1]
    return pl.pallas_call(
        _g_body,
        grid=(1,),
        in_specs=[
            pl.BlockSpec(memory_space=pltpu.SMEM),
            pl.BlockSpec((NG, do), lambda i: (0, 0)),
            pl.BlockSpec((NG, do), lambda i: (0, 0)),
            pl.BlockSpec((8, do), lambda i: (0, 0)),
            pl.BlockSpec((8, do), lambda i: (0, 0)),
            pl.BlockSpec((1, do), lambda i: (0, 0)),
            pl.BlockSpec((1, do), lambda i: (0, 0)),
        ],
        out_specs=pl.BlockSpec((NG, do), lambda i: (0, 0)),
        out_shape=jax.ShapeDtypeStruct((NG, do), jnp.float32),
    )(nvec, sp, cnt, sums, ssq, gamma.reshape(1, do), beta.reshape(1, do))


def kernel(x, edge_index, edge_attr, batch, params):
    n, in_dim = x.shape
    e = edge_attr.shape[0]
    src = edge_index[0]
    dst = edge_index[1]
    blk = 2000 if n % 2000 == 0 else n
    batch3 = batch.reshape(n // blk, 1, blk)
    nvec = jnp.full((1,), float(n), dtype=jnp.float32)

    e2s = [_edge_embed(edge_attr, p['We'], p['be']) for p in params]
    eid_b, src_b, dst_b, cnt_b = _make_bucket(n, e)(src, dst)

    wz = in_dim // 2
    z2 = jnp.stack([x[:, :wz], x[:, wz:]]).reshape(2 * n, wz)

    npad = -(-n // (NS * 8)) * (NS * 8)
    stride = npad // NS + 8
    zflats = []
    gcols = []
    for li, p in enumerate(params):
        wz = z2.shape[1]
        mp = _make_mp(n, e, wz)
        zeros = jnp.zeros((stride, wz), dtype=jnp.float32)
        agg2 = mp(z2, e2s[li], eid_b, src_b, dst_b, cnt_b, zeros)[:, :n]
        eps = p['eps'].reshape(1)
        p2, sums, ssq, sp, cnt = _mlp_call(
            eps, z2.reshape(2, n, wz), agg2,
            p['W1'], p['b1'], p['W2'], p['b2'], batch3, blk=blk)
        z2n, zflat = _bn_call(nvec, p2, sums, ssq, p['gamma'], p['beta'],
                              blk=blk)
        g = _g_call(nvec, sp, cnt, sums, ssq, p['gamma'], p['beta'])
        z2 = z2n.reshape(2 * n, z2n.shape[2])
        zflats.append(zflat)
        gcols.append(g)

    return (jnp.concatenate(zflats, axis=1), jnp.concatenate(gcols, axis=1))


# speedup vs baseline: 1.0333x; 1.0333x over previous
import functools

import jax
import jax.numpy as jnp
from jax import lax
from jax.experimental import pallas as pl
from jax.experimental.pallas import tpu as pltpu
from jax.experimental.pallas import tpu_sc as plsc

NC = 2
NS = 16
LN = 16
NG = 64


def _edge_embed_body(ea_ref, we_ref, be_ref, out_ref):
    w = out_ref.shape[2]
    res = jnp.dot(ea_ref[...], we_ref[...],
                  preferred_element_type=jnp.float32) + be_ref[...]
    out_ref[0] = res[:, :w]
    out_ref[1] = res[:, w:]


def _edge_embed(edge_attr, we, be, blk=None):
    e, ed = edge_attr.shape
    blk = blk or (2000 if e % 2000 == 0 else e)
    di = we.shape[1]
    w = di // 2
    out = pl.pallas_call(
        _edge_embed_body,
        grid=(e // blk,),
        in_specs=[
            pl.BlockSpec((blk, ed), lambda i: (i, 0)),
            pl.BlockSpec((ed, di), lambda i: (0, 0)),
            pl.BlockSpec((1, di), lambda i: (0, 0)),
        ],
        out_specs=pl.BlockSpec((2, blk, w), lambda i: (0, i, 0)),
        out_shape=jax.ShapeDtypeStruct((2, e, w), jnp.float32),
    )(edge_attr, we, be.reshape(1, di))
    return out.reshape(2 * e, w)


_FB = 2304
_BS = 2000


def _bucket_cap(e):
    return e // 2 + 4096


def _make_bucket(n, e):
    npad = -(-n // (NS * 8)) * (NS * 8)
    npt = npad // NS
    half = e // 2
    cap = _bucket_cap(e)
    nog = half // _BS
    mesh = plsc.VectorSubcoreMesh(core_axis_name="c", subcore_axis_name="s",
                                  num_cores=NC, num_subcores=NS)

    @functools.partial(
        pl.kernel,
        out_type=[
            jax.ShapeDtypeStruct((2 * NS * cap,), jnp.int32),
            jax.ShapeDtypeStruct((2 * NS * cap,), jnp.int32),
            jax.ShapeDtypeStruct((2 * NS * cap,), jnp.int32),
            jax.ShapeDtypeStruct((2 * NS, LN), jnp.int32),
        ],
        mesh=mesh,
        scratch_types=[
            pltpu.VMEM((_BS,), jnp.int32),
            pltpu.VMEM((_BS,), jnp.int32),
            pltpu.VMEM((_FB,), jnp.int32),
            pltpu.VMEM((_FB,), jnp.int32),
            pltpu.VMEM((_FB,), jnp.int32),
            pltpu.VMEM((LN,), jnp.int32),
        ],
        compiler_params=pltpu.CompilerParams(use_tc_tiling_on_sc=False,
                                             needs_layout_passes=False),
    )
    def bucket(srch, dsth, eido, srco, dsto, cnto,
               sbuf, dbuf, eidl, srcl, dstl, cbuf):
        cid = lax.axis_index("c")
        sid = lax.axis_index("s")
        lo = sid * npt
        hi = lo + npt

        ebase = cid * half
        region = (cid * NS + sid) * cap
        iota = lax.iota(jnp.int32, LN)

        def flush3(written):
            off = pl.multiple_of(region + written, 2048)
            pltpu.sync_copy(eidl.at[pl.ds(0, 2048)],
                            eido.at[pl.ds(off, 2048)])
            pltpu.sync_copy(srcl.at[pl.ds(0, 2048)],
                            srco.at[pl.ds(off, 2048)])
            pltpu.sync_copy(dstl.at[pl.ds(0, 2048)],
                            dsto.at[pl.ds(off, 2048)])

        def group(g, carry):
            cnt, written = carry
            gb = ebase + g * _BS
            pltpu.sync_copy(srch.at[pl.ds(gb, _BS)], sbuf)
            pltpu.sync_copy(dsth.at[pl.ds(gb, _BS)], dbuf)

            def step(i, carry2):
                cnt2, written2 = carry2
                vd = dbuf[pl.ds(i * LN, LN)]
                vs = sbuf[pl.ds(i * LN, LN)]
                m = jnp.logical_and(vd >= lo, vd < hi)
                pc = plsc.cumsum(m.astype(jnp.int32))
                pos = cnt2 + pc - 1
                eidv = gb + i * LN + iota
                plsc.store_scatter(eidl, [pos], eidv, mask=m)
                plsc.store_scatter(srcl, [pos], vs, mask=m)
                plsc.store_scatter(dstl, [pos], vd - lo, mask=m)
                cnt2 = cnt2 + pc[LN - 1]

                @pl.when(cnt2 >= 2048)
                def _():
                    flush3(written2)
                    eidl[pl.ds(0, LN)] = eidl[pl.ds(2048, LN)]
                    srcl[pl.ds(0, LN)] = srcl[pl.ds(2048, LN)]
                    dstl[pl.ds(0, LN)] = dstl[pl.ds(2048, LN)]

                spill = cnt2 >= 2048
                cnt2 = jnp.where(spill, cnt2 - 2048, cnt2)
                written2 = jnp.where(spill, written2 + 2048, written2)
                return (cnt2, written2)

            return lax.fori_loop(0, _BS // LN, step, (cnt, written))

        cnt, written = lax.fori_loop(0, nog, group, (0, 0))
        dummy_d = jnp.full((LN,), npt, jnp.int32)
        zero_v = jnp.zeros((LN,), jnp.int32)
        for j in range(128 // LN):
            off = cnt + j * LN + iota
            plsc.store_scatter(eidl, [off], zero_v)
            plsc.store_scatter(srcl, [off], zero_v)
            plsc.store_scatter(dstl, [off], dummy_d)
        flush3(written)

        @pl.when(cnt + 128 > 2048)
        def _():
            off2 = pl.multiple_of(region + written + 2048, 2048)
            pltpu.sync_copy(eidl.at[pl.ds(2048, 256)],
                            eido.at[pl.ds(off2, 256)])
            pltpu.sync_copy(srcl.at[pl.ds(2048, 256)],
                            srco.at[pl.ds(off2, 256)])
            pltpu.sync_copy(dstl.at[pl.ds(2048, 256)],
                            dsto.at[pl.ds(off2, 256)])

        cbuf[pl.ds(0, LN)] = jnp.full((LN,), written + cnt, jnp.int32)
        pltpu.sync_copy(cbuf, cnto.at[cid * NS + sid])

    return bucket


def _make_mp(n, e, w):
    npad = -(-n // (NS * 8)) * (NS * 8)
    npt = npad // NS
    stride = npt + 8
    cap = _bucket_cap(e)
    _CH = 80 if w >= 128 else 128
    mesh = plsc.VectorSubcoreMesh(core_axis_name="c", subcore_axis_name="s",
                                  num_cores=NC, num_subcores=NS)

    @functools.partial(
        pl.kernel,
        out_type=jax.ShapeDtypeStruct((NC, npad, w), jnp.float32),
        mesh=mesh,
        scratch_types=[
            pltpu.VMEM((2, _CH), jnp.int32),
            pltpu.VMEM((2, _CH), jnp.int32),
            pltpu.VMEM((2, _CH), jnp.int32),
            pltpu.VMEM((2, _CH), jnp.int32),
            pltpu.VMEM((2, LN), jnp.int32),
            pltpu.VMEM((2, _CH, w), jnp.float32),
            pltpu.VMEM((2, _CH, w), jnp.float32),
            pltpu.VMEM_SHARED((NS * (npt + 8), w), jnp.float32),
            pltpu.SemaphoreType.DMA,
            pltpu.SemaphoreType.DMA,
            pltpu.SemaphoreType.DMA,
            pltpu.SemaphoreType.DMA,
            pltpu.SemaphoreType.DMA,
        ],
        compiler_params=pltpu.CompilerParams(use_tc_tiling_on_sc=False),
    )
    def mp(z2, e2, eidh, srch, dsth, cnth, zeros, out,
           eidb, srcb, dstb, dsts, cbuf, zbuf, ebuf, acc,
           semi0, semi1, semg0, semg1, sems):
        cid = lax.axis_index("c")
        sid = lax.axis_index("s")
        arow = sid * stride
        pltpu.sync_copy(zeros, acc.at[pl.ds(arow, stride)])
        zoff = cid * n
        eoff = cid * e

        pltpu.sync_copy(cnth.at[sid], cbuf.at[0])
        pltpu.sync_copy(cnth.at[NS + sid], cbuf.at[1])
        cnt0 = cbuf[0, pl.ds(0, LN)][0]
        cnt1 = cbuf[1, pl.ds(0, LN)][0]
        nch0 = (cnt0 + (_CH - 1)) // _CH
        nch1 = (cnt1 + (_CH - 1)) // _CH
        total = nch0 + nch1
        base0 = sid * cap
        base1 = (NS + sid) * cap
        semi = (semi0, semi1)
        semg = (semg0, semg1)

        def off_of(ci):
            return jnp.where(ci < nch0, base0 + ci * _CH,
                             base1 + (ci - nch0) * _CH)

        def issue_idx(ci, p):
            o = pl.multiple_of(off_of(ci), 8)
            pltpu.async_copy(eidh.at[pl.ds(o, _CH)], eidb.at[p], semi[p])
            pltpu.async_copy(srch.at[pl.ds(o, _CH)], srcb.at[p], semi[p])
            pltpu.async_copy(dsth.at[pl.ds(o, _CH)], dstb.at[p], semi[p])

        def wait_idx(p):
            pltpu.make_async_copy(eidh.at[pl.ds(0, _CH)], eidb.at[p],
                                  semi[p]).wait()
            pltpu.make_async_copy(srch.at[pl.ds(0, _CH)], srcb.at[p],
                                  semi[p]).wait()
            pltpu.make_async_copy(dsth.at[pl.ds(0, _CH)], dstb.at[p],
                                  semi[p]).wait()

        def adjust_and_gather(p):
            for k in range(_CH // LN):
                sl = pl.ds(k * LN, LN)
                srcb[p, sl] = srcb[p, sl] + zoff
                eidb[p, sl] = eidb[p, sl] + eoff
            pltpu.async_copy(z2.at[srcb.at[p]], zbuf.at[p], semg[p])
            pltpu.async_copy(e2.at[eidb.at[p]], ebuf.at[p], semg[p])

        def wait_gather(p):
            pltpu.make_async_copy(z2.at[pl.ds(0, _CH)], zbuf.at[p],
                                  semg[p]).wait()
            pltpu.make_async_copy(e2.at[pl.ds(0, _CH)], ebuf.at[p],
                                  semg[p]).wait()

        def compute(p):
            def rowfn(rr, rc):
                for k in range(w // LN):
                    sl = pl.ds(k * LN, LN)
                    v = zbuf[p, rr, sl] + ebuf[p, rr, sl]
                    ebuf[p, rr, sl] = jnp.maximum(v, 0.0)
                return rc
            lax.fori_loop(0, _CH, rowfn, 0)

        def save_dst(p):
            for k in range(_CH // LN):
                sl = pl.ds(k * LN, LN)
                dsts[p, sl] = dstb[p, sl] + arow

        def issue_scatter(p):
            pltpu.async_copy(ebuf.at[p], acc.at[dsts.at[p]], sems, add=True)

        def drain_scatter():
            pltpu.make_async_copy(ebuf.at[0], acc.at[pl.ds(0, _CH)],
                                  sems).wait()

        @pl.when(total > 0)
        def _():
            issue_idx(0, 0)
            wait_idx(0)
            adjust_and_gather(0)

            @pl.when(total > 1)
            def _():
                issue_idx(1, 1)

            def pair(j, carry):
                i0 = 2 * j

                def stage(i, p, q):
                    @pl.when(i >= 1)
                    def _():
                        drain_scatter()

                    @pl.when(i + 1 < total)
                    def _():
                        wait_idx(q)
                        adjust_and_gather(q)
                    wait_gather(p)
                    save_dst(p)

                    @pl.when(i + 2 < total)
                    def _():
                        issue_idx(i + 2, p)
                    compute(p)
                    issue_scatter(p)

                @pl.when(i0 < total)
                def _():
                    stage(i0, 0, 1)

                @pl.when(i0 + 1 < total)
                def _():
                    stage(i0 + 1, 1, 0)
                return carry

            lax.fori_loop(0, (total + 1) // 2, pair, 0)
            drain_scatter()

        pltpu.sync_copy(acc.at[pl.ds(arow, npt)],
                        out.at[cid, pl.ds(sid * npt, npt)])

    return mp


def _mlp_body(eps_ref, z2_ref, agg2_ref, w1_ref, b1_ref, w2_ref, b2_ref,
              batch_ref, p2_ref, sums_ref, ssq_ref, sp_ref, cnt_ref):
    i = pl.program_id(0)
    wo = p2_ref.shape[2]
    z = jnp.concatenate([z2_ref[0], z2_ref[1]], axis=1)
    agg = jnp.concatenate([agg2_ref[0], agg2_ref[1]], axis=1)
    h = (1.0 + eps_ref[0]) * z + agg
    h = jnp.maximum(
        jnp.dot(h, w1_ref[...], preferred_element_type=jnp.float32)
        + b1_ref[...], 0.0)
    h = jnp.dot(h, w2_ref[...], preferred_element_type=jnp.float32) + b2_ref[...]
    p = jnp.maximum(h, 0.0)
    p2_ref[0] = p[:, :wo]
    p2_ref[1] = p[:, wo:]

    blk = z.shape[0]
    bvec = batch_ref[0]
    giota = lax.broadcasted_iota(jnp.int32, (NG, blk), 0)
    onehot = (giota == bvec).astype(jnp.float32)

    @pl.when(i == 0)
    def _():
        sums_ref[...] = jnp.zeros_like(sums_ref)
        ssq_ref[...] = jnp.zeros_like(ssq_ref)
        sp_ref[...] = jnp.zeros_like(sp_ref)
        cnt_ref[...] = jnp.zeros_like(cnt_ref)

    do = p.shape[1]
    sums_ref[...] += jnp.broadcast_to(jnp.sum(p, 0, keepdims=True), (8, do))
    ssq_ref[...] += jnp.broadcast_to(jnp.sum(p * p, 0, keepdims=True), (8, do))
    sp_ref[...] += jnp.dot(onehot, p, preferred_element_type=jnp.float32)
    cnt_ref[...] += jnp.broadcast_to(
        jnp.sum(onehot, 1, keepdims=True), (NG, do))


def _mlp_call(eps, z2, agg2, w1, b1, w2, b2, batch3, blk=2000):
    n = z2.shape[1]
    wz = z2.shape[2]
    do = w1.shape[1]
    wo = do // 2
    di = w1.shape[0]
    grid = (n // blk,)
    return pl.pallas_call(
        _mlp_body,
        grid=grid,
        in_specs=[
            pl.BlockSpec(memory_space=pltpu.SMEM),
            pl.BlockSpec((2, blk, wz), lambda i: (0, i, 0)),
            pl.BlockSpec((2, blk, wz), lambda i: (0, i, 0)),
            pl.BlockSpec((di, do), lambda i: (0, 0)),
            pl.BlockSpec((1, do), lambda i: (0, 0)),
            pl.BlockSpec((do, do), lambda i: (0, 0)),
            pl.BlockSpec((1, do), lambda i: (0, 0)),
            pl.BlockSpec((1, 1, blk), lambda i: (i, 0, 0)),
        ],
        out_specs=[
            pl.BlockSpec((2, blk, wo), lambda i: (0, i, 0)),
            pl.BlockSpec((8, do), lambda i: (0, 0)),
            pl.BlockSpec((8, do), lambda i: (0, 0)),
            pl.BlockSpec((NG, do), lambda i: (0, 0)),
            pl.BlockSpec((NG, do), lambda i: (0, 0)),
        ],
        out_shape=[
            jax.ShapeDtypeStruct((2, n, wo), jnp.float32),
            jax.ShapeDtypeStruct((8, do), jnp.float32),
            jax.ShapeDtypeStruct((8, do), jnp.float32),
            jax.ShapeDtypeStruct((NG, do), jnp.float32),
            jax.ShapeDtypeStruct((NG, do), jnp.float32),
        ],
    )(eps, z2, agg2, w1, b1.reshape(1, do), w2, b2.reshape(1, do), batch3)


def _bn_body(n_ref, p2_ref, sums_ref, ssq_ref, gamma_ref, beta_ref,
             z2_ref, zf_ref):
    wo = p2_ref.shape[2]
    p = jnp.concatenate([p2_ref[0], p2_ref[1]], axis=1)
    n = n_ref[0]
    mean = sums_ref[0:1] / n
    var = ssq_ref[0:1] / n - mean * mean
    scale = lax.rsqrt(var + 1e-5) * gamma_ref[...]
    zz = (p - mean) * scale + beta_ref[...]
    z2_ref[0] = zz[:, :wo]
    z2_ref[1] = zz[:, wo:]
    zf_ref[...] = zz


def _bn_call(nvec, p2, sums, ssq, gamma, beta, blk=2000):
    n = p2.shape[1]
    wo = p2.shape[2]
    do = 2 * wo
    return pl.pallas_call(
        _bn_body,
        grid=(n // blk,),
        in_specs=[
            pl.BlockSpec(memory_space=pltpu.SMEM),
            pl.BlockSpec((2, blk, wo), lambda i: (0, i, 0)),
            pl.BlockSpec((8, do), lambda i: (0, 0)),
            pl.BlockSpec((8, do), lambda i: (0, 0)),
            pl.BlockSpec((1, do), lambda i: (0, 0)),
            pl.BlockSpec((1, do), lambda i: (0, 0)),
        ],
        out_specs=[
            pl.BlockSpec((2, blk, wo), lambda i: (0, i, 0)),
            pl.BlockSpec((blk, do), lambda i: (i, 0)),
        ],
        out_shape=[
            jax.ShapeDtypeStruct((2, n, wo), jnp.float32),
            jax.ShapeDtypeStruct((n, do), jnp.float32),
        ],
    )(nvec, p2, sums, ssq, gamma.reshape(1, do), beta.reshape(1, do))


def _g_body(n_ref, sp_ref, cnt_ref, sums_ref, ssq_ref, gamma_ref, beta_ref,
            g_ref):
    n = n_ref[0]
    mean = sums_ref[0:1] / n
    var = ssq_ref[0:1] / n - mean * mean
    scale = lax.rsqrt(var + 1e-5) * gamma_ref[...]
    cnt = cnt_ref[...]
    g_ref[...] = (sp_ref[...] - cnt * mean) * scale + cnt * beta_ref[...]


def _g_call(nvec, sp, cnt, sums, ssq, gamma, beta):
    do = sp.shape[1]
    return pl.pallas_call(
        _g_body,
        grid=(1,),
        in_specs=[
            pl.BlockSpec(memory_space=pltpu.SMEM),
            pl.BlockSpec((NG, do), lambda i: (0, 0)),
            pl.BlockSpec((NG, do), lambda i: (0, 0)),
            pl.BlockSpec((8, do), lambda i: (0, 0)),
            pl.BlockSpec((8, do), lambda i: (0, 0)),
            pl.BlockSpec((1, do), lambda i: (0, 0)),
            pl.BlockSpec((1, do), lambda i: (0, 0)),
        ],
        out_specs=pl.BlockSpec((NG, do), lambda i: (0, 0)),
        out_shape=jax.ShapeDtypeStruct((NG, do), jnp.float32),
    )(nvec, sp, cnt, sums, ssq, gamma.reshape(1, do), beta.reshape(1, do))


def kernel(x, edge_index, edge_attr, batch, params):
    n, in_dim = x.shape
    e = edge_attr.shape[0]
    src = edge_index[0]
    dst = edge_index[1]
    blk = 2000 if n % 2000 == 0 else n
    batch3 = batch.reshape(n // blk, 1, blk)
    nvec = jnp.full((1,), float(n), dtype=jnp.float32)

    e2s = [_edge_embed(edge_attr, p['We'], p['be']) for p in params]
    eid_b, src_b, dst_b, cnt_b = _make_bucket(n, e)(src, dst)

    wz = in_dim // 2
    z2 = jnp.stack([x[:, :wz], x[:, wz:]]).reshape(2 * n, wz)

    npad = -(-n // (NS * 8)) * (NS * 8)
    stride = npad // NS + 8
    zflats = []
    gcols = []
    for li, p in enumerate(params):
        wz = z2.shape[1]
        mp = _make_mp(n, e, wz)
        zeros = jnp.zeros((stride, wz), dtype=jnp.float32)
        agg2 = mp(z2, e2s[li], eid_b, src_b, dst_b, cnt_b, zeros)[:, :n]
        eps = p['eps'].reshape(1)
        p2, sums, ssq, sp, cnt = _mlp_call(
            eps, z2.reshape(2, n, wz), agg2,
            p['W1'], p['b1'], p['W2'], p['b2'], batch3, blk=blk)
        z2n, zflat = _bn_call(nvec, p2, sums, ssq, p['gamma'], p['beta'],
                              blk=blk)
        g = _g_call(nvec, sp, cnt, sums, ssq, p['gamma'], p['beta'])
        z2 = z2n.reshape(2 * n, z2n.shape[2])
        zflats.append(zflat)
        gcols.append(g)

    return (jnp.concatenate(zflats, axis=1), jnp.concatenate(gcols, axis=1))
